# baseline (speedup 1.0000x reference)
import jax
import jax.numpy as jnp
import numpy as np
from jax.experimental import pallas as pl
from jax.experimental.pallas import tpu as pltpu

_M = 10
_SLAB = 64
_CLAMP = 80.0
_LOG1EM7 = float(np.log(1e-7))
_BB = 256
_SUB = 2


def _repack(l_ref, base, t_ref, d, tm, rr):
    per_chunk = 128 // d
    for t in range(rr):
        b0 = base + t * per_chunk
        piece = l_ref[b0:b0 + per_chunk]
        piece = piece.reshape(128, tm)
        t_ref[:, t, :] = jnp.transpose(piece, (1, 0))


def _compute(t_ref, x_ref, base, halfv, lognbm1h, nb_ok, rr):
    partial = jnp.zeros((1, 128), jnp.float32)
    for s in range(rr // _SLAB):
        sl = slice(s * _SLAB, (s + 1) * _SLAB)
        xsl = slice(base + s * _SLAB, base + (s + 1) * _SLAB)
        xs = x_ref[xsl, :]
        is_lo = xs < -0.9999
        is_hi = xs > 0.9999

        rm_s = acc_s = rm_l = acc_l = None
        for m in range(_M):
            logit = t_ref[m, sl, :]
            mu = t_ref[_M + m, sl, :]
            lsc = jnp.maximum(t_ref[2 * _M + m, sl, :], -7.0)
            inv = jnp.exp(-lsc)
            c = xs - mu
            a = inv * c
            h2 = inv * halfv
            pin = a + h2
            mnn = a - h2

            ep = jnp.exp(jnp.minimum(-pin, _CLAMP))
            em = jnp.exp(jnp.minimum(-mnn, _CLAMP))
            lup = jnp.log(1.0 + ep)
            lum = jnp.log(1.0 + em)
            ldelta = jnp.log(em - ep) - lup - lum
            emid = jnp.exp(jnp.minimum(-a, _CLAMP))
            lmid = jnp.log(1.0 + emid)
            pdfmid = jnp.where(a < -_CLAMP, a, -a - 2.0 * lmid) - lsc
            lcp = jnp.where(pin < -_CLAMP, pin, -lup)
            lom = jnp.where(mnn < -_CLAMP, 0.0, -mnn - lum)
            inner = jnp.where(ldelta > _LOG1EM7, ldelta, pdfmid - lognbm1h)
            lpb = jnp.where(is_lo, lcp, jnp.where(is_hi, lom, inner))
            lpb = jnp.where(nb_ok, lpb, 0.0)
            sv = lpb + logit

            if m == 0:
                rm_s, acc_s = sv, jnp.ones_like(sv)
                rm_l, acc_l = logit, jnp.ones_like(logit)
            else:
                nm = jnp.maximum(rm_s, sv)
                acc_s = acc_s * jnp.exp(rm_s - nm) + jnp.exp(sv - nm)
                rm_s = nm
                nl = jnp.maximum(rm_l, logit)
                acc_l = acc_l * jnp.exp(rm_l - nl) + jnp.exp(logit - nl)
                rm_l = nl

        mixture = (rm_s + jnp.log(acc_s)) - (rm_l + jnp.log(acc_l))
        partial = partial + jnp.sum(mixture, axis=0, keepdims=True)
    return partial


def _dml_block(l_ref, x_ref, half_ref, lognbm1h_ref, nbgt_ref, out_ref,
               t0_ref, t1_ref):
    j = pl.program_id(0)
    d = l_ref.shape[1]
    tm = l_ref.shape[2]
    rr = _BB * d // 128

    halfv = half_ref[...]
    lognbm1h = lognbm1h_ref[...]
    nb_ok = nbgt_ref[...] > 0.5

    _repack(l_ref, 0, t0_ref, d, tm, rr)
    _repack(l_ref, _BB, t1_ref, d, tm, rr)
    p0 = _compute(t0_ref, x_ref, 0, halfv, lognbm1h, nb_ok, rr)
    p1 = _compute(t1_ref, x_ref, rr, halfv, lognbm1h, nb_ok, rr)
    partial = p0 + p1

    @pl.when(j == 0)
    def _():
        out_ref[...] = jnp.zeros_like(out_ref)

    out_ref[...] += partial[None]


@jax.jit
def kernel(x, l, input_bins, mask):
    del mask
    b, d, tm = l.shape
    rows = b * d // 128
    x2 = x.reshape(rows, 128)

    reps = 128 // d
    nb = input_bins.astype(jnp.float32)
    nbm1 = jnp.maximum(nb - 1.0, 1.0)
    halfv = jnp.tile(1.0 / nbm1, reps).reshape(1, 128)
    lognbm1h = jnp.tile(jnp.log(nbm1 / 2.0), reps).reshape(1, 128)
    nbgt = jnp.tile(jnp.where(nb > 1.5, 1.0, 0.0), reps).reshape(1, 128)

    step = _SUB * _BB
    nj = b // step
    rr = _BB * d // 128

    out = pl.pallas_call(
        _dml_block,
        grid=(nj,),
        in_specs=[
            pl.BlockSpec((step, d, tm), lambda j: (j, 0, 0)),
            pl.BlockSpec((_SUB * rr, 128), lambda j: (j, 0)),
            pl.BlockSpec((1, 128), lambda j: (0, 0)),
            pl.BlockSpec((1, 128), lambda j: (0, 0)),
            pl.BlockSpec((1, 128), lambda j: (0, 0)),
        ],
        out_specs=pl.BlockSpec((1, 1, 128), lambda j: (0, 0, 0)),
        out_shape=jax.ShapeDtypeStruct((1, 1, 128), jnp.float32),
        scratch_shapes=[pltpu.VMEM((tm, rr, 128), jnp.float32),
                        pltpu.VMEM((tm, rr, 128), jnp.float32)],
        compiler_params=pltpu.CompilerParams(
            dimension_semantics=("arbitrary",),
            vmem_limit_bytes=50 * 1024 * 1024,
        ),
        name="dml_loss",
    )(l, x2, halfv, lognbm1h, nbgt)

    neg = -jnp.sum(out)
    return neg, neg / (b * np.float32(np.log(2.0)))
